# Initial kernel scaffold; baseline (speedup 1.0000x reference)
import functools

import jax
import jax.numpy as jnp
from jax import lax
from jax.experimental import pallas as pl
from jax.experimental.pallas import tpu as pltpu
from jax.experimental.pallas import tpu_sc as plsc

_NC = 2
_NS = 16
_NW = _NC * _NS

_BATCH = 4096
_SEQ = 200
_DIM = 32
_HALF = _SEQ // 2
_ROWS_PER_W = _BATCH // _NW
_CHUNKS_PER_W = 2 * _ROWS_PER_W


def _pool_body(x_hbm, table_hbm, out_hbm, idx_v, buf_a, buf_b, sums_v,
               sem_a, sem_b):
  wid = lax.axis_index("s") * _NC + lax.axis_index("c")
  row_base = wid * _ROWS_PER_W
  chunk_base = wid * _CHUNKS_PER_W

  pltpu.sync_copy(x_hbm.at[pl.ds(chunk_base, _CHUNKS_PER_W)], idx_v)

  def fire(r, buf, sem):
    pltpu.async_copy(table_hbm.at[idx_v.at[2 * r]],
                     buf.at[pl.ds(0, _HALF)], sem)
    pltpu.async_copy(table_hbm.at[idx_v.at[2 * r + 1]],
                     buf.at[pl.ds(_HALF, _HALF)], sem)

  def wait(buf, sem):
    pltpu.make_async_copy(table_hbm.at[pl.ds(0, _SEQ)], buf, sem).wait()

  def reduce_store(buf, r):
    def body(j, c):
      a0, a1, b0, b1 = c
      p = 2 * j
      a0 = a0 + buf[p, 0:16]
      a1 = a1 + buf[p, 16:32]
      b0 = b0 + buf[p + 1, 0:16]
      b1 = b1 + buf[p + 1, 16:32]
      return (a0, a1, b0, b1)

    z = jnp.zeros((16,), jnp.float32)
    a0, a1, b0, b1 = lax.fori_loop(0, _HALF, body, (z, z, z, z), unroll=4)
    sums_v[r, 0:16] = a0 + b0
    sums_v[r, 16:32] = a1 + b1

  fire(0, buf_a, sem_a)

  def outer(i, carry):
    r0 = 2 * i
    r1 = r0 + 1
    fire(r1, buf_b, sem_b)
    wait(buf_a, sem_a)
    reduce_store(buf_a, r0)

    @pl.when(i < _ROWS_PER_W // 2 - 1)
    def _():
      fire(r0 + 2, buf_a, sem_a)

    wait(buf_b, sem_b)
    reduce_store(buf_b, r1)
    return carry

  lax.fori_loop(0, _ROWS_PER_W // 2, outer, 0)

  pltpu.sync_copy(sums_v, out_hbm.at[pl.ds(row_base, _ROWS_PER_W)])


_pool = functools.partial(
    pl.kernel,
    out_type=jax.ShapeDtypeStruct((_BATCH, _DIM), jnp.float32),
    mesh=plsc.VectorSubcoreMesh(core_axis_name="c", subcore_axis_name="s"),
    scratch_types=[
        pltpu.VMEM((_CHUNKS_PER_W, _HALF), jnp.int32),
        pltpu.VMEM((_SEQ, _DIM), jnp.float32),
        pltpu.VMEM((_SEQ, _DIM), jnp.float32),
        pltpu.VMEM((_ROWS_PER_W, _DIM), jnp.float32),
        pltpu.SemaphoreType.DMA,
        pltpu.SemaphoreType.DMA,
    ],
)(_pool_body)


def _head_body(s_ref, l_ref, wt_ref, b_ref, o_ref):
  s = s_ref[...]
  l = l_ref[...].astype(jnp.float32)
  rep = jnp.maximum(s / l, 0.0)
  o_ref[...] = (
      jnp.dot(rep, wt_ref[...], preferred_element_type=jnp.float32)
      + b_ref[...])


@jax.jit
def _head(sums, lengths, wt, bias):
  return pl.pallas_call(
      _head_body,
      out_shape=jax.ShapeDtypeStruct((_BATCH, 10), jnp.float32),
  )(sums, lengths, wt, bias)


@jax.jit
def kernel(x, lengths, emb_table, W, b):
  x2 = x.reshape(_BATCH * 2, _HALF)
  sums = _pool(x2, emb_table)
  return _head(sums, lengths.reshape(_BATCH, 1), W.T, b.reshape(1, -1))


# speedup vs baseline: 2.2799x; 2.2799x over previous
import functools

import jax
import jax.numpy as jnp
from jax import lax
from jax.experimental import pallas as pl
from jax.experimental.pallas import tpu as pltpu
from jax.experimental.pallas import tpu_sc as plsc

_NC = 2
_NS = 16
_NW = _NC * _NS

_BATCH = 4096
_SEQ = 200
_DIM = 32
_HALF = _SEQ // 2
_ROWS_PER_W = _BATCH // _NW
_CHUNKS_PER_W = 2 * _ROWS_PER_W


def _pool_body(x_hbm, table_hbm, out_hbm, idx_v, buf_a, buf_b, sums_v,
               sem_a, sem_b):
  wid = lax.axis_index("s") * _NC + lax.axis_index("c")
  row_base = wid * _ROWS_PER_W
  chunk_base = wid * _CHUNKS_PER_W

  pltpu.sync_copy(x_hbm.at[pl.ds(chunk_base, _CHUNKS_PER_W)], idx_v)

  def fire(r, buf, sem):
    pltpu.async_copy(table_hbm.at[idx_v.at[2 * r]],
                     buf.at[pl.ds(0, _HALF)], sem)
    pltpu.async_copy(table_hbm.at[idx_v.at[2 * r + 1]],
                     buf.at[pl.ds(_HALF, _HALF)], sem)

  def wait(buf, sem):
    pltpu.make_async_copy(table_hbm.at[pl.ds(0, _SEQ)], buf, sem).wait()

  def reduce_store(buf, r):
    def body(j, c):
      a0, a1, b0, b1 = c
      p = 2 * j
      a0 = a0 + buf[p, 0:16]
      a1 = a1 + buf[p, 16:32]
      b0 = b0 + buf[p + 1, 0:16]
      b1 = b1 + buf[p + 1, 16:32]
      return (a0, a1, b0, b1)

    z = jnp.zeros((16,), jnp.float32)
    a0, a1, b0, b1 = lax.fori_loop(0, _HALF, body, (z, z, z, z), unroll=4)
    sums_v[r, 0:16] = a0 + b0
    sums_v[r, 16:32] = a1 + b1

  fire(0, buf_a, sem_a)

  def outer(i, carry):
    r0 = 2 * i
    r1 = r0 + 1
    fire(r1, buf_b, sem_b)
    wait(buf_a, sem_a)
    reduce_store(buf_a, r0)

    @pl.when(i < _ROWS_PER_W // 2 - 1)
    def _():
      fire(r0 + 2, buf_a, sem_a)

    wait(buf_b, sem_b)
    reduce_store(buf_b, r1)
    return carry

  lax.fori_loop(0, _ROWS_PER_W // 2, outer, 0)

  pltpu.sync_copy(sums_v, out_hbm.at[pl.ds(row_base, _ROWS_PER_W)])


_pool = functools.partial(
    pl.kernel,
    out_type=jax.ShapeDtypeStruct((_BATCH, _DIM), jnp.float32),
    mesh=plsc.VectorSubcoreMesh(core_axis_name="c", subcore_axis_name="s"),
    scratch_types=[
        pltpu.VMEM((_CHUNKS_PER_W, _HALF), jnp.int32),
        pltpu.VMEM((_SEQ, _DIM), jnp.float32),
        pltpu.VMEM((_SEQ, _DIM), jnp.float32),
        pltpu.VMEM((_ROWS_PER_W, _DIM), jnp.float32),
        pltpu.SemaphoreType.DMA,
        pltpu.SemaphoreType.DMA,
    ],
    compiler_params=pltpu.CompilerParams(use_tc_tiling_on_sc=False),
)(_pool_body)


def _head_body(s_ref, l_ref, wt_ref, b_ref, o_ref):
  s = s_ref[...]
  l = l_ref[...].astype(jnp.float32)
  rep = jnp.maximum(s / l, 0.0)
  o_ref[...] = (
      jnp.dot(rep, wt_ref[...], preferred_element_type=jnp.float32)
      + b_ref[...])


@jax.jit
def _head(sums, lengths, wt, bias):
  return pl.pallas_call(
      _head_body,
      out_shape=jax.ShapeDtypeStruct((_BATCH, 10), jnp.float32),
  )(sums, lengths, wt, bias)


@jax.jit
def kernel(x, lengths, emb_table, W, b):
  x2 = x.reshape(_BATCH * 2, _HALF)
  sums = _pool(x2, emb_table)
  return _head(sums, lengths.reshape(_BATCH, 1), W.T, b.reshape(1, -1))
